# baseline (speedup 1.0000x reference)
import functools

import jax
import jax.numpy as jnp
from jax import lax
from jax.experimental import pallas as pl
from jax.experimental.pallas import tpu as pltpu
from jax.experimental.pallas import tpu_sc as plsc

EMBED = 64
HIDDEN = 128
L = 16384

NC = 2
NS = 16
NW = NC * NS
PER_W = L // NW
CHUNK = 128
NCHUNK = PER_W // CHUNK
VREGS = EMBED // 16


@functools.partial(
    pl.kernel,
    mesh=plsc.VectorSubcoreMesh(core_axis_name="c", subcore_axis_name="s"),
    out_type=jax.ShapeDtypeStruct((NW, EMBED), jnp.float32),
    compiler_params=pltpu.CompilerParams(use_tc_tiling_on_sc=False),
    scratch_types=[
        pltpu.VMEM((NCHUNK, CHUNK), jnp.int32),
        pltpu.VMEM((NCHUNK, CHUNK, EMBED), jnp.float32),
        pltpu.VMEM((EMBED,), jnp.float32),
        pltpu.SemaphoreType.DMA,
    ],
)
def _gather_sum(x_hbm, table_hbm, out_hbm, idx_v, rows_v, acc_v, sem):
    wid = lax.axis_index("s") * NC + lax.axis_index("c")
    pltpu.sync_copy(x_hbm.at[wid], idx_v)
    copies = [
        pltpu.async_copy(table_hbm.at[idx_v.at[j]], rows_v.at[j], sem)
        for j in range(NCHUNK)
    ]
    for c in copies:
        c.wait()

    zero = jnp.zeros((16,), jnp.float32)

    def body(i, accs):
        out = []
        for cix in range(NCHUNK):
            for v in range(VREGS):
                out.append(accs[cix * VREGS + v]
                           + rows_v[cix, i, pl.ds(v * 16, 16)])
        return tuple(out)

    accs = lax.fori_loop(0, CHUNK, body, (zero,) * (NCHUNK * VREGS))
    for v in range(VREGS):
        total = accs[v]
        for cix in range(1, NCHUNK):
            total = total + accs[cix * VREGS + v]
        acc_v[pl.ds(v * 16, 16)] = total
    pltpu.sync_copy(acc_v, out_hbm.at[wid])


def _mlp_body(p_ref, w1_ref, b1_ref, w2_ref, b2_ref, o_ref):
    e = jnp.sum(p_ref[...], axis=0, keepdims=True) * (1.0 / L)
    h = lax.dot_general(e, w1_ref[...], (((1,), (1,)), ((), ())),
                        preferred_element_type=jnp.float32)
    h = jnp.maximum(h + b1_ref[...], 0.0)
    o_ref[...] = jnp.sum(h * w2_ref[...], axis=1, keepdims=True) + b2_ref[...]


def kernel(x, table, W1, b1, W2, b2):
    xi = x.astype(jnp.int32).reshape(NW, NCHUNK, CHUNK)
    partials = _gather_sum(xi, table)
    out = pl.pallas_call(
        _mlp_body,
        out_shape=jax.ShapeDtypeStruct((1, 1), jnp.float32),
    )(partials, W1, b1.reshape(1, HIDDEN), W2, b2.reshape(1, 1))
    return out.reshape(1)


# speedup vs baseline: 5.4685x; 5.4685x over previous
import functools

import jax
import jax.numpy as jnp
from jax import lax
from jax.experimental import pallas as pl
from jax.experimental.pallas import tpu as pltpu
from jax.experimental.pallas import tpu_sc as plsc

VOCAB = 1000000
EMBED = 64
HIDDEN = 128
L = 16384

NC = 2
NS = 16
NW = NC * NS
PER_W = L // NW
CHUNK = 128
NCHUNK = PER_W // CHUNK

B = 32768
G = 31
P = G * B
SLICE = P // NS


@functools.partial(
    pl.kernel,
    mesh=plsc.VectorSubcoreMesh(core_axis_name="c", subcore_axis_name="s"),
    out_type=jax.ShapeDtypeStruct((NC, P), jnp.float32),
    scratch_types=[
        pltpu.VMEM((NCHUNK, CHUNK), jnp.int32),
        pltpu.VMEM((NCHUNK, CHUNK), jnp.float32),
        pltpu.VMEM_SHARED((P,), jnp.float32),
    ],
)
def _sc_counts(x_hbm, zeros_hbm, out_hbm, idx_v, ones_v, c_sh):
    cid = lax.axis_index("c")
    sid = lax.axis_index("s")
    wid = sid * NC + cid
    pltpu.sync_copy(x_hbm.at[wid], idx_v)
    for j in range(NCHUNK):
        for k in range(CHUNK // 16):
            ones_v[j, pl.ds(k * 16, 16)] = jnp.ones((16,), jnp.float32)
    pltpu.sync_copy(zeros_hbm.at[pl.ds(sid * SLICE, SLICE)],
                    c_sh.at[pl.ds(sid * SLICE, SLICE)])
    plsc.subcore_barrier()
    for j in range(NCHUNK):
        pltpu.sync_copy(ones_v.at[j], c_sh.at[idx_v.at[j]], add=True)
    plsc.subcore_barrier()
    pltpu.sync_copy(c_sh.at[pl.ds(sid * SLICE, SLICE)],
                    out_hbm.at[cid, pl.ds(sid * SLICE, SLICE)])


def _matvec_body(tab_ref, cnt_ref, w1_ref, b1_ref, w2_ref, b2_ref, o_ref,
                 acc_ref):
    i = pl.program_id(0)

    @pl.when(i == 0)
    def _():
        acc_ref[...] = jnp.zeros_like(acc_ref)

    lane = lax.broadcasted_iota(jnp.int32, (1, B), 1) + i * B
    tb = jnp.where(lane < VOCAB, tab_ref[...], 0.0)
    c = cnt_ref[0:1, :] + cnt_ref[1:2, :]
    acc_ref[...] += lax.dot_general(tb, c, (((1,), (1,)), ((), ())),
                                    preferred_element_type=jnp.float32)

    @pl.when(i == G - 1)
    def _():
        e = acc_ref[...] * (1.0 / L)
        h = lax.dot_general(w1_ref[...], e, (((1,), (0,)), ((), ())),
                            preferred_element_type=jnp.float32)
        h = jnp.maximum(h + b1_ref[...], 0.0)
        o_ref[...] = lax.dot_general(w2_ref[...], h, (((1,), (0,)), ((), ())),
                                     preferred_element_type=jnp.float32) \
            + b2_ref[...]


def kernel(x, table, W1, b1, W2, b2):
    xi = x.astype(jnp.int32).reshape(NW, NCHUNK, CHUNK)
    zeros = jnp.zeros((P,), jnp.float32)
    counts = _sc_counts(xi, zeros)
    tableT = jnp.transpose(table)
    out = pl.pallas_call(
        _matvec_body,
        grid=(G,),
        in_specs=[
            pl.BlockSpec((EMBED, B), lambda i: (0, i)),
            pl.BlockSpec((NC, B), lambda i: (0, i)),
            pl.BlockSpec((HIDDEN, EMBED), lambda i: (0, 0)),
            pl.BlockSpec((HIDDEN, 1), lambda i: (0, 0)),
            pl.BlockSpec((1, HIDDEN), lambda i: (0, 0)),
            pl.BlockSpec((1, 1), lambda i: (0, 0)),
        ],
        out_specs=pl.BlockSpec((1, 1), lambda i: (0, 0)),
        out_shape=jax.ShapeDtypeStruct((1, 1), jnp.float32),
        scratch_shapes=[pltpu.VMEM((EMBED, 1), jnp.float32)],
    )(tableT, counts, W1, b1.reshape(HIDDEN, 1), W2, b2.reshape(1, 1))
    return out.reshape(1)


# speedup vs baseline: 5.5494x; 1.0148x over previous
import functools

import jax
import jax.numpy as jnp
from jax import lax
from jax.experimental import pallas as pl
from jax.experimental.pallas import tpu as pltpu
from jax.experimental.pallas import tpu_sc as plsc

VOCAB = 1000000
EMBED = 64
HIDDEN = 128
L = 16384

NC = 2
NS = 16
NW = NC * NS
PER_W = L // NW
CHUNK = 128
NCHUNK = PER_W // CHUNK

B = 49152
G = 21
P = G * B
SLICE = P // NS


@functools.partial(
    pl.kernel,
    mesh=plsc.VectorSubcoreMesh(core_axis_name="c", subcore_axis_name="s"),
    out_type=jax.ShapeDtypeStruct((NC, P), jnp.float32),
    scratch_types=[
        pltpu.VMEM((NCHUNK, CHUNK), jnp.int32),
        pltpu.VMEM((NCHUNK, CHUNK), jnp.float32),
        pltpu.VMEM_SHARED((P,), jnp.float32),
    ],
)
def _sc_counts(x_hbm, zeros_hbm, out_hbm, idx_v, ones_v, c_sh):
    cid = lax.axis_index("c")
    sid = lax.axis_index("s")
    wid = sid * NC + cid
    pltpu.sync_copy(x_hbm.at[wid], idx_v)
    for j in range(NCHUNK):
        for k in range(CHUNK // 16):
            ones_v[j, pl.ds(k * 16, 16)] = jnp.ones((16,), jnp.float32)
    pltpu.sync_copy(zeros_hbm.at[pl.ds(sid * SLICE, SLICE)],
                    c_sh.at[pl.ds(sid * SLICE, SLICE)])
    plsc.subcore_barrier()
    for j in range(NCHUNK):
        pltpu.sync_copy(ones_v.at[j], c_sh.at[idx_v.at[j]], add=True)
    plsc.subcore_barrier()
    pltpu.sync_copy(c_sh.at[pl.ds(sid * SLICE, SLICE)],
                    out_hbm.at[cid, pl.ds(sid * SLICE, SLICE)])


def _matvec_body(tab_ref, cnt_ref, w1_ref, b1_ref, w2_ref, b2_ref, o_ref,
                 acc_ref):
    i = pl.program_id(0)

    @pl.when(i == 0)
    def _():
        acc_ref[...] = jnp.zeros_like(acc_ref)

    c = cnt_ref[0:1, :] + cnt_ref[1:2, :]

    @pl.when(i < G - 1)
    def _():
        acc_ref[...] += lax.dot_general(tab_ref[...], c,
                                        (((1,), (1,)), ((), ())),
                                        preferred_element_type=jnp.float32)

    @pl.when(i == G - 1)
    def _():
        lane = lax.broadcasted_iota(jnp.int32, (1, B), 1) + i * B
        tb = jnp.where(lane < VOCAB, tab_ref[...], 0.0)
        acc_ref[...] += lax.dot_general(tb, c, (((1,), (1,)), ((), ())),
                                        preferred_element_type=jnp.float32)

    @pl.when(i == G - 1)
    def _():
        e = acc_ref[...] * (1.0 / L)
        h = lax.dot_general(w1_ref[...], e, (((1,), (0,)), ((), ())),
                            preferred_element_type=jnp.float32)
        h = jnp.maximum(h + b1_ref[...], 0.0)
        o_ref[...] = lax.dot_general(w2_ref[...], h, (((1,), (0,)), ((), ())),
                                     preferred_element_type=jnp.float32) \
            + b2_ref[...]


def kernel(x, table, W1, b1, W2, b2):
    xi = x.astype(jnp.int32).reshape(NW, NCHUNK, CHUNK)
    zeros = jnp.zeros((P,), jnp.float32)
    counts = _sc_counts(xi, zeros)
    tableT = jnp.transpose(table)
    out = pl.pallas_call(
        _matvec_body,
        grid=(G,),
        in_specs=[
            pl.BlockSpec((EMBED, B), lambda i: (0, i)),
            pl.BlockSpec((NC, B), lambda i: (0, i)),
            pl.BlockSpec((HIDDEN, EMBED), lambda i: (0, 0)),
            pl.BlockSpec((HIDDEN, 1), lambda i: (0, 0)),
            pl.BlockSpec((1, HIDDEN), lambda i: (0, 0)),
            pl.BlockSpec((1, 1), lambda i: (0, 0)),
        ],
        out_specs=pl.BlockSpec((1, 1), lambda i: (0, 0)),
        out_shape=jax.ShapeDtypeStruct((1, 1), jnp.float32),
        scratch_shapes=[pltpu.VMEM((EMBED, 1), jnp.float32)],
    )(tableT, counts, W1, b1.reshape(HIDDEN, 1), W2, b2.reshape(1, 1))
    return out.reshape(1)
